# baseline (speedup 1.0000x reference)
import functools

import jax
import jax.numpy as jnp
from jax import lax
from jax.experimental import pallas as pl
from jax.experimental.pallas import tpu as pltpu
from jax.experimental.pallas import tpu_sc as plsc

_NC = 2
_NS = 16
_NW = _NC * _NS
_CH = 128


def _make_sc_layer1(NP, CPW):
  RPT = NP // _NS
  mesh = plsc.VectorSubcoreMesh(core_axis_name="c", subcore_axis_name="s")

  @functools.partial(
      pl.kernel,
      mesh=mesh,
      compiler_params=pltpu.CompilerParams(use_tc_tiling_on_sc=False),
      out_type=[
          jax.ShapeDtypeStruct((_NC, NP, 4, 16), jnp.float32),
          jax.ShapeDtypeStruct((_NC, NP, 16), jnp.float32),
      ],
      scratch_types=[
          pltpu.VMEM((_CH,), jnp.int32),
          pltpu.VMEM((_CH,), jnp.int32),
          pltpu.VMEM((_CH, 16), jnp.float32),
          pltpu.VMEM((_CH, 16), jnp.float32),
          pltpu.VMEM((_CH, 4, 16), jnp.float32),
          pltpu.VMEM((_CH,), jnp.int32),
          pltpu.VMEM((_CH,), jnp.int32),
          pltpu.VMEM((_CH, 16), jnp.float32),
          pltpu.VMEM((_CH, 16), jnp.float32),
          pltpu.VMEM((_CH, 4, 16), jnp.float32),
          pltpu.VMEM((_CH, 16), jnp.float32),
          pltpu.VMEM((_CH, 4, 16), jnp.float32),
          pltpu.VMEM_SHARED((NP, 4, 16), jnp.float32),
          pltpu.VMEM_SHARED((NP, 16), jnp.float32),
          pltpu.SemaphoreType.DMA,
          pltpu.SemaphoreType.DMA,
      ],
  )
  def k(src_h, dst_h, ts_h, td_h, hh_h, z64_h, z16_h, outp, denp,
        src_v0, dst_v0, ts_v0, td_v0, h_v0,
        src_v1, dst_v1, ts_v1, td_v1, h_v1,
        ex_v, msg_v, out_sh, den_sh, sem0, sem1):
    cid = lax.axis_index("c")
    sid = lax.axis_index("s")

    @pl.when(sid == 0)
    def _init():
      pltpu.sync_copy(z64_h, out_sh)
      pltpu.sync_copy(z16_h, den_sh)

    plsc.subcore_barrier()
    wid = sid * _NC + cid
    bufs = ((src_v0, dst_v0, ts_v0, td_v0, h_v0, sem0),
            (src_v1, dst_v1, ts_v1, td_v1, h_v1, sem1))

    def fire(q, bi):
      src_v, dst_v, ts_v, td_v, h_v, sem = bufs[bi]
      base = (wid * CPW + q) * _CH
      c1 = pltpu.async_copy(src_h.at[pl.ds(base, _CH)], src_v, sem)
      c2 = pltpu.async_copy(dst_h.at[pl.ds(base, _CH)], dst_v, sem)
      c1.wait()
      c2.wait()
      pltpu.async_copy(ts_h.at[src_v], ts_v, sem)
      pltpu.async_copy(td_h.at[dst_v], td_v, sem)
      pltpu.async_copy(hh_h.at[src_v], h_v, sem)

    def drain(bi):
      src_v, dst_v, ts_v, td_v, h_v, sem = bufs[bi]
      pltpu.make_async_copy(ts_h.at[pl.ds(0, _CH)], ts_v, sem).wait()
      pltpu.make_async_copy(td_h.at[pl.ds(0, _CH)], td_v, sem).wait()
      pltpu.make_async_copy(hh_h.at[pl.ds(0, _CH)], h_v, sem).wait()

    def compute(bi):
      src_v, dst_v, ts_v, td_v, h_v, sem = bufs[bi]

      def edge_body(e, c2):
        al = ts_v[e] + td_v[e]
        al = jnp.maximum(al, al * 0.2)
        ex = jnp.exp(al)
        m8 = jnp.arange(16, dtype=jnp.int32) < 8
        ex = jnp.where(m8, ex, 0.0)
        ex_v[e] = ex
        for kk in range(4):
          b0 = jnp.full((16,), ex[2 * kk], dtype=jnp.float32)
          b1 = jnp.full((16,), ex[2 * kk + 1], dtype=jnp.float32)
          ev = jnp.where(m8, b0, b1)
          msg_v[e, kk] = ev * h_v[e, kk]
        return c2

      lax.fori_loop(0, _CH, edge_body, 0, unroll=4)
      pltpu.sync_copy(msg_v, out_sh.at[dst_v], add=True)
      pltpu.sync_copy(ex_v, den_sh.at[dst_v], add=True)

    fire(0, 0)
    pairs, rem = divmod(CPW - 1, 2)

    def pair_body(jp, carry):
      q0 = jp * 2
      fire(q0 + 1, 1)
      drain(0)
      compute(0)
      fire(q0 + 2, 0)
      drain(1)
      compute(1)
      return carry

    lax.fori_loop(0, pairs, pair_body, 0)
    q = pairs * 2
    if rem:
      fire(q + 1, (q + 1) % 2)
      drain(q % 2)
      compute(q % 2)
      q += 1
    drain(q % 2)
    compute(q % 2)
    plsc.subcore_barrier()
    pltpu.sync_copy(out_sh.at[pl.ds(sid * RPT, RPT)],
                    outp.at[cid, pl.ds(sid * RPT, RPT)])
    pltpu.sync_copy(den_sh.at[pl.ds(sid * RPT, RPT)],
                    denp.at[cid, pl.ds(sid * RPT, RPT)])

  return k


def _make_sc_layer2(NP, CPW):
  RPT = NP // _NS
  mesh = plsc.VectorSubcoreMesh(core_axis_name="c", subcore_axis_name="s")

  @functools.partial(
      pl.kernel,
      mesh=mesh,
      compiler_params=pltpu.CompilerParams(use_tc_tiling_on_sc=False),
      out_type=[jax.ShapeDtypeStruct((_NC, NP, 16), jnp.float32)],
      scratch_types=[
          pltpu.VMEM((_CH,), jnp.int32),
          pltpu.VMEM((_CH,), jnp.int32),
          pltpu.VMEM((_CH, 16), jnp.float32),
          pltpu.VMEM((_CH, 16), jnp.float32),
          pltpu.VMEM((_CH, 16), jnp.float32),
          pltpu.VMEM((_CH,), jnp.int32),
          pltpu.VMEM((_CH,), jnp.int32),
          pltpu.VMEM((_CH, 16), jnp.float32),
          pltpu.VMEM((_CH, 16), jnp.float32),
          pltpu.VMEM((_CH, 16), jnp.float32),
          pltpu.VMEM((_CH, 16), jnp.float32),
          pltpu.VMEM_SHARED((NP, 16), jnp.float32),
          pltpu.SemaphoreType.DMA,
          pltpu.SemaphoreType.DMA,
      ],
  )
  def k(src_h, dst_h, t2s_h, t2d_h, g2_h, z16_h, outp,
        src_v0, dst_v0, ts_v0, td_v0, g_v0,
        src_v1, dst_v1, ts_v1, td_v1, g_v1,
        msg_v, out_sh, sem0, sem1):
    cid = lax.axis_index("c")
    sid = lax.axis_index("s")

    @pl.when(sid == 0)
    def _init():
      pltpu.sync_copy(z16_h, out_sh)

    plsc.subcore_barrier()
    wid = sid * _NC + cid
    bufs = ((src_v0, dst_v0, ts_v0, td_v0, g_v0, sem0),
            (src_v1, dst_v1, ts_v1, td_v1, g_v1, sem1))

    def fire(q, bi):
      src_v, dst_v, ts_v, td_v, g_v, sem = bufs[bi]
      base = (wid * CPW + q) * _CH
      c1 = pltpu.async_copy(src_h.at[pl.ds(base, _CH)], src_v, sem)
      c2 = pltpu.async_copy(dst_h.at[pl.ds(base, _CH)], dst_v, sem)
      c1.wait()
      c2.wait()
      pltpu.async_copy(t2s_h.at[src_v], ts_v, sem)
      pltpu.async_copy(t2d_h.at[dst_v], td_v, sem)
      pltpu.async_copy(g2_h.at[src_v], g_v, sem)

    def drain(bi):
      src_v, dst_v, ts_v, td_v, g_v, sem = bufs[bi]
      pltpu.make_async_copy(t2s_h.at[pl.ds(0, _CH)], ts_v, sem).wait()
      pltpu.make_async_copy(t2d_h.at[pl.ds(0, _CH)], td_v, sem).wait()
      pltpu.make_async_copy(g2_h.at[pl.ds(0, _CH)], g_v, sem).wait()

    def compute(bi):
      src_v, dst_v, ts_v, td_v, g_v, sem = bufs[bi]

      def edge_body(e, c2):
        al = ts_v[e] + td_v[e]
        al = jnp.maximum(al, al * 0.2)
        ex = jnp.exp(al)
        ev = jnp.full((16,), ex[0], dtype=jnp.float32)
        msg_v[e] = ev * g_v[e]
        return c2

      lax.fori_loop(0, _CH, edge_body, 0, unroll=4)
      pltpu.sync_copy(msg_v, out_sh.at[dst_v], add=True)

    fire(0, 0)
    pairs, rem = divmod(CPW - 1, 2)

    def pair_body(jp, carry):
      q0 = jp * 2
      fire(q0 + 1, 1)
      drain(0)
      compute(0)
      fire(q0 + 2, 0)
      drain(1)
      compute(1)
      return carry

    lax.fori_loop(0, pairs, pair_body, 0)
    q = pairs * 2
    if rem:
      fire(q + 1, (q + 1) % 2)
      drain(q % 2)
      compute(q % 2)
      q += 1
    drain(q % 2)
    compute(q % 2)
    plsc.subcore_barrier()
    pltpu.sync_copy(out_sh.at[pl.ds(sid * RPT, RPT)],
                    outp.at[cid, pl.ds(sid * RPT, RPT)])

  return k


def _tc_prep(x_p, W1, As16, Ad16, NP):
  BLK = 1024

  def body(x_ref, w_ref, as_ref, ad_ref, ts_ref, td_ref, h_ref):
    h = jnp.dot(x_ref[...], w_ref[...], preferred_element_type=jnp.float32)
    h_ref[...] = h
    ts_ref[...] = jnp.dot(h, as_ref[...], preferred_element_type=jnp.float32)
    td_ref[...] = jnp.dot(h, ad_ref[...], preferred_element_type=jnp.float32)

  return pl.pallas_call(
      body,
      grid=(NP // BLK,),
      in_specs=[
          pl.BlockSpec((BLK, 128), lambda i: (i, 0)),
          pl.BlockSpec((128, 64), lambda i: (0, 0)),
          pl.BlockSpec((64, 16), lambda i: (0, 0)),
          pl.BlockSpec((64, 16), lambda i: (0, 0)),
      ],
      out_specs=[
          pl.BlockSpec((BLK, 16), lambda i: (i, 0)),
          pl.BlockSpec((BLK, 16), lambda i: (i, 0)),
          pl.BlockSpec((BLK, 64), lambda i: (i, 0)),
      ],
      out_shape=[
          jax.ShapeDtypeStruct((NP, 16), jnp.float32),
          jax.ShapeDtypeStruct((NP, 16), jnp.float32),
          jax.ShapeDtypeStruct((NP, 64), jnp.float32),
      ],
  )(x_p, W1, As16, Ad16)


def _tc_mid(outp, denp, B16, b1r, W2s16, W2d16, W2g16, NP):
  BLK = 1024

  def body(o_ref, d_ref, b16_ref, b1_ref, ws_ref, wd_ref, wg_ref,
           t2s_ref, t2d_ref, g2_ref):
    o = o_ref[0] + o_ref[1]
    d = d_ref[0] + d_ref[1]
    dmat = jnp.dot(d, b16_ref[...], preferred_element_type=jnp.float32)
    z = o / (dmat + 1e-16) + b1_ref[...]
    h1 = jnp.where(z > 0, z, jnp.exp(z) - 1.0)
    t2s_ref[...] = jnp.dot(h1, ws_ref[...], preferred_element_type=jnp.float32)
    t2d_ref[...] = jnp.dot(h1, wd_ref[...], preferred_element_type=jnp.float32)
    g2 = jnp.dot(h1, wg_ref[...], preferred_element_type=jnp.float32)
    col = lax.broadcasted_iota(jnp.int32, (g2.shape[0], 16), 1)
    g2_ref[...] = jnp.where(col == 1, 1.0, g2)

  return pl.pallas_call(
      body,
      grid=(NP // BLK,),
      in_specs=[
          pl.BlockSpec((2, BLK, 64), lambda i: (0, i, 0)),
          pl.BlockSpec((2, BLK, 16), lambda i: (0, i, 0)),
          pl.BlockSpec((16, 64), lambda i: (0, 0)),
          pl.BlockSpec((1, 64), lambda i: (0, 0)),
          pl.BlockSpec((64, 16), lambda i: (0, 0)),
          pl.BlockSpec((64, 16), lambda i: (0, 0)),
          pl.BlockSpec((64, 16), lambda i: (0, 0)),
      ],
      out_specs=[
          pl.BlockSpec((BLK, 16), lambda i: (i, 0)),
          pl.BlockSpec((BLK, 16), lambda i: (i, 0)),
          pl.BlockSpec((BLK, 16), lambda i: (i, 0)),
      ],
      out_shape=[
          jax.ShapeDtypeStruct((NP, 16), jnp.float32),
          jax.ShapeDtypeStruct((NP, 16), jnp.float32),
          jax.ShapeDtypeStruct((NP, 16), jnp.float32),
      ],
  )(outp, denp, B16, b1r, W2s16, W2d16, W2g16)


def _tc_final(out2p, b2r, NP):
  BLK = 1024

  def body(o_ref, b2_ref, res_ref):
    o = o_ref[0] + o_ref[1]
    num = o[:, 0:1]
    den = o[:, 1:2]
    z = num / (den + 1e-16) + b2_ref[...]
    res = 1.0 / (1.0 + jnp.exp(-z))
    res_ref[...] = jnp.broadcast_to(res, (res.shape[0], 16))

  return pl.pallas_call(
      body,
      grid=(NP // BLK,),
      in_specs=[
          pl.BlockSpec((2, BLK, 16), lambda i: (0, i, 0)),
          pl.BlockSpec((1, 1), lambda i: (0, 0)),
      ],
      out_specs=pl.BlockSpec((BLK, 16), lambda i: (i, 0)),
      out_shape=jax.ShapeDtypeStruct((NP, 16), jnp.float32),
  )(out2p, b2r)


@jax.jit
def kernel(x, edge_index, W1, att_src1, att_dst1, b1, W2, att_src2,
           att_dst2, b2):
  N, D = x.shape
  E = edge_index.shape[1]
  HEADS = att_src1.shape[1]
  HID = att_src1.shape[2]
  HH = HEADS * HID

  NP = ((N + 1 + 2047) // 2048) * 2048
  CPW = -(-E // (_NW * _CH))
  E_pad = _NW * _CH * CPW

  x_p = jnp.pad(x, ((0, NP - N), (0, 0)))
  src_p = jnp.concatenate(
      [edge_index[0], jnp.zeros((E_pad - E,), jnp.int32)])
  dst_p = jnp.concatenate(
      [edge_index[1], jnp.full((E_pad - E,), N, jnp.int32)])

  r64 = jnp.arange(HH)
  As16 = jnp.zeros((HH, 16), jnp.float32).at[r64, r64 // HID].set(
      att_src1[0].reshape(HH))
  Ad16 = jnp.zeros((HH, 16), jnp.float32).at[r64, r64 // HID].set(
      att_dst1[0].reshape(HH))
  B16 = jnp.zeros((16, HH), jnp.float32).at[r64 // HID, r64].set(1.0)
  b1r = b1.reshape(1, HH)
  W2s16 = jnp.pad(W2 * att_src2[0, 0, 0], ((0, 0), (0, 15)))
  W2d16 = jnp.pad(W2 * att_dst2[0, 0, 0], ((0, 0), (0, 15)))
  W2g16 = jnp.pad(W2, ((0, 0), (0, 15)))
  b2r = b2.reshape(1, 1)
  z64 = jnp.zeros((NP, 4, 16), jnp.float32)
  z16 = jnp.zeros((NP, 16), jnp.float32)

  ts, td, h = _tc_prep(x_p, W1, As16, Ad16, NP)
  h4 = h.reshape(NP, 4, 16)
  outp, denp = _make_sc_layer1(NP, CPW)(
      src_p, dst_p, ts, td, h4, z64, z16)
  outp = outp.reshape(_NC, NP, HH)

  t2s, t2d, g2 = _tc_mid(outp, denp, B16, b1r, W2s16, W2d16, W2g16, NP)

  (out2p,) = _make_sc_layer2(NP, CPW)(src_p, dst_p, t2s, t2d, g2, z16)

  res = _tc_final(out2p, b2r, NP)
  return res[:N, 0:1]


# speedup vs baseline: 1.1533x; 1.1533x over previous
import functools

import jax
import jax.numpy as jnp
from jax import lax
from jax.experimental import pallas as pl
from jax.experimental.pallas import tpu as pltpu
from jax.experimental.pallas import tpu_sc as plsc

_NC = 2
_NS = 16
_NW = _NC * _NS
_CH = 128


def _make_sc_layer1(NP, CPW):
  RPT = NP // _NS
  mesh = plsc.VectorSubcoreMesh(core_axis_name="c", subcore_axis_name="s")

  @functools.partial(
      pl.kernel,
      mesh=mesh,
      compiler_params=pltpu.CompilerParams(use_tc_tiling_on_sc=False),
      out_type=[
          jax.ShapeDtypeStruct((_NC, NP, 4, 16), jnp.float32),
          jax.ShapeDtypeStruct((_NC, NP, 16), jnp.float32),
      ],
      scratch_types=[
          pltpu.VMEM((_CH,), jnp.int32),
          pltpu.VMEM((_CH,), jnp.int32),
          pltpu.VMEM((_CH, 16), jnp.float32),
          pltpu.VMEM((_CH, 16), jnp.float32),
          pltpu.VMEM((_CH, 4, 16), jnp.float32),
          pltpu.VMEM((_CH,), jnp.int32),
          pltpu.VMEM((_CH,), jnp.int32),
          pltpu.VMEM((_CH, 16), jnp.float32),
          pltpu.VMEM((_CH, 16), jnp.float32),
          pltpu.VMEM((_CH, 4, 16), jnp.float32),
          pltpu.VMEM((_CH, 16), jnp.float32),
          pltpu.VMEM((_CH, 4, 16), jnp.float32),
          pltpu.VMEM_SHARED((NP, 4, 16), jnp.float32),
          pltpu.VMEM_SHARED((NP, 16), jnp.float32),
          pltpu.SemaphoreType.DMA,
          pltpu.SemaphoreType.DMA,
      ],
  )
  def k(src_h, dst_h, ts_h, td_h, hh_h, z64_h, z16_h, outp, denp,
        src_v0, dst_v0, ts_v0, td_v0, h_v0,
        src_v1, dst_v1, ts_v1, td_v1, h_v1,
        ex_v, msg_v, out_sh, den_sh, sem0, sem1):
    cid = lax.axis_index("c")
    sid = lax.axis_index("s")

    @pl.when(sid == 0)
    def _init():
      pltpu.sync_copy(z64_h, out_sh)
      pltpu.sync_copy(z16_h, den_sh)

    plsc.subcore_barrier()
    wid = sid * _NC + cid
    bufs = ((src_v0, dst_v0, ts_v0, td_v0, h_v0, sem0),
            (src_v1, dst_v1, ts_v1, td_v1, h_v1, sem1))

    def fire(q, bi):
      src_v, dst_v, ts_v, td_v, h_v, sem = bufs[bi]
      base = (wid * CPW + q) * _CH
      c1 = pltpu.async_copy(src_h.at[pl.ds(base, _CH)], src_v, sem)
      c2 = pltpu.async_copy(dst_h.at[pl.ds(base, _CH)], dst_v, sem)
      c1.wait()
      c2.wait()
      pltpu.async_copy(ts_h.at[src_v], ts_v, sem)
      pltpu.async_copy(td_h.at[dst_v], td_v, sem)
      pltpu.async_copy(hh_h.at[src_v], h_v, sem)

    def drain(bi):
      src_v, dst_v, ts_v, td_v, h_v, sem = bufs[bi]
      pltpu.make_async_copy(ts_h.at[pl.ds(0, _CH)], ts_v, sem).wait()
      pltpu.make_async_copy(td_h.at[pl.ds(0, _CH)], td_v, sem).wait()
      pltpu.make_async_copy(hh_h.at[pl.ds(0, _CH)], h_v, sem).wait()

    def compute(bi):
      src_v, dst_v, ts_v, td_v, h_v, sem = bufs[bi]

      def edge_body(e, c2):
        al = ts_v[e] + td_v[e]
        al = jnp.maximum(al, al * 0.2)
        ex = jnp.exp(al)
        m8 = jnp.arange(16, dtype=jnp.int32) < 8
        ex = jnp.where(m8, ex, 0.0)
        ex_v[e] = ex
        for kk in range(4):
          b0 = jnp.full((16,), ex[2 * kk], dtype=jnp.float32)
          b1 = jnp.full((16,), ex[2 * kk + 1], dtype=jnp.float32)
          ev = jnp.where(m8, b0, b1)
          msg_v[e, kk] = ev * h_v[e, kk]
        return c2

      lax.fori_loop(0, _CH, edge_body, 0)
      pltpu.sync_copy(msg_v, out_sh.at[dst_v], add=True)
      pltpu.sync_copy(ex_v, den_sh.at[dst_v], add=True)

    fire(0, 0)
    pairs, rem = divmod(CPW - 1, 2)

    def pair_body(jp, carry):
      q0 = jp * 2
      fire(q0 + 1, 1)
      drain(0)
      compute(0)
      fire(q0 + 2, 0)
      drain(1)
      compute(1)
      return carry

    lax.fori_loop(0, pairs, pair_body, 0)
    q = pairs * 2
    if rem:
      fire(q + 1, (q + 1) % 2)
      drain(q % 2)
      compute(q % 2)
      q += 1
    drain(q % 2)
    compute(q % 2)
    plsc.subcore_barrier()
    pltpu.sync_copy(out_sh.at[pl.ds(sid * RPT, RPT)],
                    outp.at[cid, pl.ds(sid * RPT, RPT)])
    pltpu.sync_copy(den_sh.at[pl.ds(sid * RPT, RPT)],
                    denp.at[cid, pl.ds(sid * RPT, RPT)])

  return k


def _make_sc_layer2(NP, CPW):
  RPT = NP // _NS
  mesh = plsc.VectorSubcoreMesh(core_axis_name="c", subcore_axis_name="s")

  @functools.partial(
      pl.kernel,
      mesh=mesh,
      compiler_params=pltpu.CompilerParams(use_tc_tiling_on_sc=False),
      out_type=[jax.ShapeDtypeStruct((_NC, NP, 16), jnp.float32)],
      scratch_types=[
          pltpu.VMEM((_CH,), jnp.int32),
          pltpu.VMEM((_CH,), jnp.int32),
          pltpu.VMEM((_CH, 16), jnp.float32),
          pltpu.VMEM((_CH, 16), jnp.float32),
          pltpu.VMEM((_CH, 16), jnp.float32),
          pltpu.VMEM((_CH,), jnp.int32),
          pltpu.VMEM((_CH,), jnp.int32),
          pltpu.VMEM((_CH, 16), jnp.float32),
          pltpu.VMEM((_CH, 16), jnp.float32),
          pltpu.VMEM((_CH, 16), jnp.float32),
          pltpu.VMEM((_CH, 16), jnp.float32),
          pltpu.VMEM_SHARED((NP, 16), jnp.float32),
          pltpu.SemaphoreType.DMA,
          pltpu.SemaphoreType.DMA,
      ],
  )
  def k(src_h, dst_h, t2s_h, t2d_h, g2_h, z16_h, outp,
        src_v0, dst_v0, ts_v0, td_v0, g_v0,
        src_v1, dst_v1, ts_v1, td_v1, g_v1,
        msg_v, out_sh, sem0, sem1):
    cid = lax.axis_index("c")
    sid = lax.axis_index("s")

    @pl.when(sid == 0)
    def _init():
      pltpu.sync_copy(z16_h, out_sh)

    plsc.subcore_barrier()
    wid = sid * _NC + cid
    bufs = ((src_v0, dst_v0, ts_v0, td_v0, g_v0, sem0),
            (src_v1, dst_v1, ts_v1, td_v1, g_v1, sem1))

    def fire(q, bi):
      src_v, dst_v, ts_v, td_v, g_v, sem = bufs[bi]
      base = (wid * CPW + q) * _CH
      c1 = pltpu.async_copy(src_h.at[pl.ds(base, _CH)], src_v, sem)
      c2 = pltpu.async_copy(dst_h.at[pl.ds(base, _CH)], dst_v, sem)
      c1.wait()
      c2.wait()
      pltpu.async_copy(t2s_h.at[src_v], ts_v, sem)
      pltpu.async_copy(t2d_h.at[dst_v], td_v, sem)
      pltpu.async_copy(g2_h.at[src_v], g_v, sem)

    def drain(bi):
      src_v, dst_v, ts_v, td_v, g_v, sem = bufs[bi]
      pltpu.make_async_copy(t2s_h.at[pl.ds(0, _CH)], ts_v, sem).wait()
      pltpu.make_async_copy(t2d_h.at[pl.ds(0, _CH)], td_v, sem).wait()
      pltpu.make_async_copy(g2_h.at[pl.ds(0, _CH)], g_v, sem).wait()

    def compute(bi):
      src_v, dst_v, ts_v, td_v, g_v, sem = bufs[bi]

      def edge_body(e, c2):
        al = ts_v[e] + td_v[e]
        al = jnp.maximum(al, al * 0.2)
        ex = jnp.exp(al)
        ev = jnp.full((16,), ex[0], dtype=jnp.float32)
        msg_v[e] = ev * g_v[e]
        return c2

      lax.fori_loop(0, _CH, edge_body, 0)
      pltpu.sync_copy(msg_v, out_sh.at[dst_v], add=True)

    fire(0, 0)
    pairs, rem = divmod(CPW - 1, 2)

    def pair_body(jp, carry):
      q0 = jp * 2
      fire(q0 + 1, 1)
      drain(0)
      compute(0)
      fire(q0 + 2, 0)
      drain(1)
      compute(1)
      return carry

    lax.fori_loop(0, pairs, pair_body, 0)
    q = pairs * 2
    if rem:
      fire(q + 1, (q + 1) % 2)
      drain(q % 2)
      compute(q % 2)
      q += 1
    drain(q % 2)
    compute(q % 2)
    plsc.subcore_barrier()
    pltpu.sync_copy(out_sh.at[pl.ds(sid * RPT, RPT)],
                    outp.at[cid, pl.ds(sid * RPT, RPT)])

  return k


def _tc_prep(x_p, W1, As16, Ad16, NP):
  BLK = 1024

  def body(x_ref, w_ref, as_ref, ad_ref, ts_ref, td_ref, h_ref):
    h = jnp.dot(x_ref[...], w_ref[...], preferred_element_type=jnp.float32)
    h_ref[...] = h
    ts_ref[...] = jnp.dot(h, as_ref[...], preferred_element_type=jnp.float32)
    td_ref[...] = jnp.dot(h, ad_ref[...], preferred_element_type=jnp.float32)

  return pl.pallas_call(
      body,
      grid=(NP // BLK,),
      in_specs=[
          pl.BlockSpec((BLK, 128), lambda i: (i, 0)),
          pl.BlockSpec((128, 64), lambda i: (0, 0)),
          pl.BlockSpec((64, 16), lambda i: (0, 0)),
          pl.BlockSpec((64, 16), lambda i: (0, 0)),
      ],
      out_specs=[
          pl.BlockSpec((BLK, 16), lambda i: (i, 0)),
          pl.BlockSpec((BLK, 16), lambda i: (i, 0)),
          pl.BlockSpec((BLK, 64), lambda i: (i, 0)),
      ],
      out_shape=[
          jax.ShapeDtypeStruct((NP, 16), jnp.float32),
          jax.ShapeDtypeStruct((NP, 16), jnp.float32),
          jax.ShapeDtypeStruct((NP, 64), jnp.float32),
      ],
  )(x_p, W1, As16, Ad16)


def _tc_mid(outp, denp, B16, b1r, W2s16, W2d16, W2g16, NP):
  BLK = 1024

  def body(o_ref, d_ref, b16_ref, b1_ref, ws_ref, wd_ref, wg_ref,
           t2s_ref, t2d_ref, g2_ref):
    o = o_ref[0] + o_ref[1]
    d = d_ref[0] + d_ref[1]
    dmat = jnp.dot(d, b16_ref[...], preferred_element_type=jnp.float32)
    z = o / (dmat + 1e-16) + b1_ref[...]
    h1 = jnp.where(z > 0, z, jnp.exp(z) - 1.0)
    t2s_ref[...] = jnp.dot(h1, ws_ref[...], preferred_element_type=jnp.float32)
    t2d_ref[...] = jnp.dot(h1, wd_ref[...], preferred_element_type=jnp.float32)
    g2 = jnp.dot(h1, wg_ref[...], preferred_element_type=jnp.float32)
    col = lax.broadcasted_iota(jnp.int32, (g2.shape[0], 16), 1)
    g2_ref[...] = jnp.where(col == 1, 1.0, g2)

  return pl.pallas_call(
      body,
      grid=(NP // BLK,),
      in_specs=[
          pl.BlockSpec((2, BLK, 64), lambda i: (0, i, 0)),
          pl.BlockSpec((2, BLK, 16), lambda i: (0, i, 0)),
          pl.BlockSpec((16, 64), lambda i: (0, 0)),
          pl.BlockSpec((1, 64), lambda i: (0, 0)),
          pl.BlockSpec((64, 16), lambda i: (0, 0)),
          pl.BlockSpec((64, 16), lambda i: (0, 0)),
          pl.BlockSpec((64, 16), lambda i: (0, 0)),
      ],
      out_specs=[
          pl.BlockSpec((BLK, 16), lambda i: (i, 0)),
          pl.BlockSpec((BLK, 16), lambda i: (i, 0)),
          pl.BlockSpec((BLK, 16), lambda i: (i, 0)),
      ],
      out_shape=[
          jax.ShapeDtypeStruct((NP, 16), jnp.float32),
          jax.ShapeDtypeStruct((NP, 16), jnp.float32),
          jax.ShapeDtypeStruct((NP, 16), jnp.float32),
      ],
  )(outp, denp, B16, b1r, W2s16, W2d16, W2g16)


def _tc_final(out2p, b2r, NP):
  BLK = 1024

  def body(o_ref, b2_ref, res_ref):
    o = o_ref[0] + o_ref[1]
    num = o[:, 0:1]
    den = o[:, 1:2]
    z = num / (den + 1e-16) + b2_ref[...]
    res = 1.0 / (1.0 + jnp.exp(-z))
    res_ref[...] = jnp.broadcast_to(res, (res.shape[0], 16))

  return pl.pallas_call(
      body,
      grid=(NP // BLK,),
      in_specs=[
          pl.BlockSpec((2, BLK, 16), lambda i: (0, i, 0)),
          pl.BlockSpec((1, 1), lambda i: (0, 0)),
      ],
      out_specs=pl.BlockSpec((BLK, 16), lambda i: (i, 0)),
      out_shape=jax.ShapeDtypeStruct((NP, 16), jnp.float32),
  )(out2p, b2r)


@jax.jit
def kernel(x, edge_index, W1, att_src1, att_dst1, b1, W2, att_src2,
           att_dst2, b2):
  N, D = x.shape
  E = edge_index.shape[1]
  HEADS = att_src1.shape[1]
  HID = att_src1.shape[2]
  HH = HEADS * HID

  NP = ((N + 1 + 2047) // 2048) * 2048
  CPW = -(-E // (_NW * _CH))
  E_pad = _NW * _CH * CPW

  x_p = jnp.pad(x, ((0, NP - N), (0, 0)))
  src_p = jnp.concatenate(
      [edge_index[0], jnp.zeros((E_pad - E,), jnp.int32)])
  dst_p = jnp.concatenate(
      [edge_index[1], jnp.full((E_pad - E,), N, jnp.int32)])

  r64 = jnp.arange(HH)
  As16 = jnp.zeros((HH, 16), jnp.float32).at[r64, r64 // HID].set(
      att_src1[0].reshape(HH))
  Ad16 = jnp.zeros((HH, 16), jnp.float32).at[r64, r64 // HID].set(
      att_dst1[0].reshape(HH))
  B16 = jnp.zeros((16, HH), jnp.float32).at[r64 // HID, r64].set(1.0)
  b1r = b1.reshape(1, HH)
  W2s16 = jnp.pad(W2 * att_src2[0, 0, 0], ((0, 0), (0, 15)))
  W2d16 = jnp.pad(W2 * att_dst2[0, 0, 0], ((0, 0), (0, 15)))
  W2g16 = jnp.pad(W2, ((0, 0), (0, 15)))
  b2r = b2.reshape(1, 1)
  z64 = jnp.zeros((NP, 4, 16), jnp.float32)
  z16 = jnp.zeros((NP, 16), jnp.float32)

  ts, td, h = _tc_prep(x_p, W1, As16, Ad16, NP)
  h4 = h.reshape(NP, 4, 16)
  outp, denp = _make_sc_layer1(NP, CPW)(
      src_p, dst_p, ts, td, h4, z64, z16)
  outp = outp.reshape(_NC, NP, HH)

  t2s, t2d, g2 = _tc_mid(outp, denp, B16, b1r, W2s16, W2d16, W2g16, NP)

  (out2p,) = _make_sc_layer2(NP, CPW)(src_p, dst_p, t2s, t2d, g2, z16)

  res = _tc_final(out2p, b2r, NP)
  return res[:N, 0:1]
